# Initial kernel scaffold; baseline (speedup 1.0000x reference)
import jax
import jax.numpy as jnp
from jax.experimental import pallas as pl


def kernel(x, in_degree, out_degree, in_deg_emb, out_deg_emb, graph_token):
    raise NotImplementedError("write your pallas kernel here")


# speedup vs baseline: 2.7648x; 2.7648x over previous
import functools

import jax
import jax.numpy as jnp
from jax import lax
from jax.experimental import pallas as pl
from jax.experimental.pallas import tpu as pltpu
from jax.experimental.pallas import tpu_sc as plsc

G = 128
N = 512
D = 128
NC = 2
NS = 16
NW = NC * NS
GPW = G // NW
B = 128
NB = N // B
LANES = 16


def _body(x_hbm, ind_hbm, outd_hbm, inemb_hbm, outemb_hbm, tok_hbm, out_hbm,
          idx_in, idx_out, xv, inr, outr, tokv, sem_a, sem_b):
    wid = lax.axis_index("s") * NC + lax.axis_index("c")
    pltpu.sync_copy(tok_hbm, tokv)
    for gi in range(GPW):
        g = wid * GPW + gi
        pltpu.sync_copy(tokv, out_hbm.at[pl.ds(g * (N + 1) * D, D)])
        for blk in range(NB):
            node0 = g * N + blk * B
            pltpu.sync_copy(ind_hbm.at[pl.ds(node0, B)], idx_in)
            pltpu.sync_copy(outd_hbm.at[pl.ds(node0, B)], idx_out)
            cp_a = pltpu.async_copy(inemb_hbm.at[idx_in], inr, sem_a)
            cp_b = pltpu.async_copy(outemb_hbm.at[idx_out], outr, sem_b)
            pltpu.sync_copy(x_hbm.at[pl.ds(node0 * D, B * D)], xv)
            cp_a.wait()
            cp_b.wait()

            def body(i, carry):
                for j in range(D // LANES):
                    s = pl.ds(j * LANES, LANES)
                    f = pl.ds(i * D + j * LANES, LANES)
                    xv[f] = xv[f] + inr[i, s] + outr[i, s]
                return carry

            lax.fori_loop(0, B, body, 0)
            row0 = (g * (N + 1) + 1 + blk * B) * D
            pltpu.sync_copy(xv, out_hbm.at[pl.ds(row0, B * D)])


@jax.jit
def _run(x2, ind, outd, inemb, outemb, tok):
    mesh = plsc.VectorSubcoreMesh(core_axis_name="c", subcore_axis_name="s")
    fn = functools.partial(
        pl.kernel,
        out_type=jax.ShapeDtypeStruct((G * (N + 1) * D,), jnp.float32),
        mesh=mesh,
        compiler_params=pltpu.CompilerParams(use_tc_tiling_on_sc=False),
        scratch_types=[
            pltpu.VMEM((B,), jnp.int32),
            pltpu.VMEM((B,), jnp.int32),
            pltpu.VMEM((B * D,), jnp.float32),
            pltpu.VMEM((B, D), jnp.float32),
            pltpu.VMEM((B, D), jnp.float32),
            pltpu.VMEM((D,), jnp.float32),
            pltpu.SemaphoreType.DMA,
            pltpu.SemaphoreType.DMA,
        ],
    )(_body)
    return fn(x2, ind, outd, inemb, outemb, tok)


def kernel(x, in_degree, out_degree, in_deg_emb, out_deg_emb, graph_token):
    x2 = x.reshape(G * N * D)
    ind = in_degree.reshape(-1).astype(jnp.int32)
    outd = out_degree.reshape(-1).astype(jnp.int32)
    tok = graph_token.reshape(D)
    out = _run(x2, ind, outd, in_deg_emb, out_deg_emb, tok)
    return out.reshape(G, N + 1, D)


# speedup vs baseline: 3.2337x; 1.1696x over previous
import functools

import jax
import jax.numpy as jnp
from jax import lax
from jax.experimental import pallas as pl
from jax.experimental.pallas import tpu as pltpu
from jax.experimental.pallas import tpu_sc as plsc

G = 128
N = 512
D = 128
NC = 2
NS = 16
NW = NC * NS
GPW = G // NW
B = 128
NB = N // B
NBLK = GPW * NB
LANES = 16


def _body(x_hbm, ind_hbm, outd_hbm, inemb_hbm, outemb_hbm, tok_hbm, out_hbm,
          idx_in, idx_out, xv, inr, outr, tokv, sem_in, sem_out):
    wid = lax.axis_index("s") * NC + lax.axis_index("c")
    g0 = wid * GPW

    pltpu.sync_copy(tok_hbm, tokv)
    for gi in range(GPW):
        pltpu.sync_copy(tokv, out_hbm.at[pl.ds((g0 + gi) * (N + 1) * D, D)])

    def fetch(t, p):
        node0 = (g0 * N) + t * B
        pltpu.sync_copy(ind_hbm.at[pl.ds(node0, B)], idx_in[p])
        pltpu.sync_copy(outd_hbm.at[pl.ds(node0, B)], idx_out[p])
        return (
            pltpu.async_copy(inemb_hbm.at[idx_in[p]], inr[p], sem_in[p]),
            pltpu.async_copy(outemb_hbm.at[idx_out[p]], outr[p], sem_in[p]),
            pltpu.async_copy(x_hbm.at[pl.ds(node0 * D, B * D)], xv[p],
                             sem_in[p]),
        )

    inflight = [None, None]
    stores = [None, None]
    inflight[0] = fetch(0, 0)
    for t in range(NBLK):
        p = t % 2
        if t + 1 < NBLK:
            inflight[1 - p] = fetch(t + 1, 1 - p)
        for cp in inflight[p]:
            cp.wait()
        if stores[p] is not None:
            stores[p].wait()

        xvp, inrp, outrp = xv[p], inr[p], outr[p]

        @plsc.parallel_loop(0, B, unroll=4)
        def _(i):
            for j in range(D // LANES):
                s = pl.ds(j * LANES, LANES)
                f = pl.ds(i * D + j * LANES, LANES)
                xvp[f] = xvp[f] + inrp[i, s] + outrp[i, s]

        g, blk = divmod(t, NB)
        row0 = ((g0 + g) * (N + 1) + 1 + blk * B) * D
        stores[p] = pltpu.async_copy(
            xv[p], out_hbm.at[pl.ds(row0, B * D)], sem_out[p])
    for st in stores:
        if st is not None:
            st.wait()


@jax.jit
def _run(x2, ind, outd, inemb, outemb, tok):
    mesh = plsc.VectorSubcoreMesh(core_axis_name="c", subcore_axis_name="s")
    fn = functools.partial(
        pl.kernel,
        out_type=jax.ShapeDtypeStruct((G * (N + 1) * D,), jnp.float32),
        mesh=mesh,
        compiler_params=pltpu.CompilerParams(use_tc_tiling_on_sc=False),
        scratch_types=[
            [pltpu.VMEM((B,), jnp.int32)] * 2,
            [pltpu.VMEM((B,), jnp.int32)] * 2,
            [pltpu.VMEM((B * D,), jnp.float32)] * 2,
            [pltpu.VMEM((B, D), jnp.float32)] * 2,
            [pltpu.VMEM((B, D), jnp.float32)] * 2,
            pltpu.VMEM((D,), jnp.float32),
            [pltpu.SemaphoreType.DMA] * 2,
            [pltpu.SemaphoreType.DMA] * 2,
        ],
    )(_body)
    return fn(x2, ind, outd, inemb, outemb, tok)


def kernel(x, in_degree, out_degree, in_deg_emb, out_deg_emb, graph_token):
    x2 = x.reshape(G * N * D)
    ind = in_degree.reshape(-1).astype(jnp.int32)
    outd = out_degree.reshape(-1).astype(jnp.int32)
    tok = graph_token.reshape(D)
    out = _run(x2, ind, outd, in_deg_emb, out_deg_emb, tok)
    return out.reshape(G, N + 1, D)


# speedup vs baseline: 3.8889x; 1.2026x over previous
import functools

import jax
import jax.numpy as jnp
from jax import lax
from jax.experimental import pallas as pl
from jax.experimental.pallas import tpu as pltpu
from jax.experimental.pallas import tpu_sc as plsc

G = 128
N = 512
D = 128
NC = 2
NS = 16
NW = NC * NS
GPW = G // NW
B = 128
NB = N // B
NBLK = GPW * NB
LANES = 16
SL = D // LANES


def _body(x_hbm, ind_hbm, outd_hbm, inemb_hbm, outemb_hbm, tok_hbm, out_hbm,
          idx_in, idx_out, xv, inr, outr, ov, tokv, sem_in, sem_out,
          sem_tail):
    wid = lax.axis_index("s") * NC + lax.axis_index("c")
    g0 = wid * GPW

    pltpu.sync_copy(tok_hbm, tokv)

    def fetch(t, p):
        g, blk = divmod(t, NB)
        node0 = (g0 + g) * N + blk * B
        pltpu.sync_copy(ind_hbm.at[pl.ds(node0, B)], idx_in[p])
        pltpu.sync_copy(outd_hbm.at[pl.ds(node0, B)], idx_out[p])
        return (
            pltpu.async_copy(inemb_hbm.at[idx_in[p]], inr[p], sem_in[p]),
            pltpu.async_copy(outemb_hbm.at[idx_out[p]], outr[p], sem_in[p]),
            pltpu.async_copy(x_hbm.at[g0 + g, pl.ds(blk * B, B), :], xv[p],
                             sem_in[p]),
        )

    inflight = [None, None]
    store = None
    tail = None
    inflight[0] = fetch(0, 0)
    for t in range(NBLK):
        p = t % 2
        g, blk = divmod(t, NB)
        if t + 1 < NBLK:
            inflight[1 - p] = fetch(t + 1, 1 - p)
        for cp in inflight[p]:
            cp.wait()
        if store is not None:
            store.wait()
        if tail is not None:
            tail.wait()
            tail = None

        if blk == 0:
            for j in range(SL):
                s = pl.ds(j * LANES, LANES)
                ov[0, s] = tokv[0, s]
        else:
            for j in range(SL):
                s = pl.ds(j * LANES, LANES)
                ov[0, s] = ov[B, s]

        xvp, inrp, outrp = xv[p], inr[p], outr[p]

        @plsc.parallel_loop(0, B, unroll=4)
        def _(i):
            for j in range(SL):
                s = pl.ds(j * LANES, LANES)
                ov[i + 1, s] = xvp[i, s] + inrp[i, s] + outrp[i, s]

        store = pltpu.async_copy(
            ov.at[pl.ds(0, B)],
            out_hbm.at[g0 + g, pl.ds(blk * B, B), :], sem_out)
        if blk == NB - 1:
            tail = pltpu.async_copy(
                ov.at[pl.ds(B, 1)],
                out_hbm.at[g0 + g, pl.ds(N, 1), :], sem_tail)
    store.wait()
    tail.wait()


@jax.jit
def _run(x, ind, outd, inemb, outemb, tok):
    mesh = plsc.VectorSubcoreMesh(core_axis_name="c", subcore_axis_name="s")
    fn = functools.partial(
        pl.kernel,
        out_type=jax.ShapeDtypeStruct((G, N + 1, D), jnp.float32),
        mesh=mesh,
        scratch_types=[
            [pltpu.VMEM((B,), jnp.int32)] * 2,
            [pltpu.VMEM((B,), jnp.int32)] * 2,
            [pltpu.VMEM((B, D), jnp.float32)] * 2,
            [pltpu.VMEM((B, D), jnp.float32)] * 2,
            [pltpu.VMEM((B, D), jnp.float32)] * 2,
            pltpu.VMEM((B + 8, D), jnp.float32),
            pltpu.VMEM((1, D), jnp.float32),
            [pltpu.SemaphoreType.DMA] * 2,
            pltpu.SemaphoreType.DMA,
            pltpu.SemaphoreType.DMA,
        ],
    )(_body)
    return fn(x, ind, outd, inemb, outemb, tok)


def kernel(x, in_degree, out_degree, in_deg_emb, out_deg_emb, graph_token):
    ind = in_degree.reshape(-1).astype(jnp.int32)
    outd = out_degree.reshape(-1).astype(jnp.int32)
    return _run(x, ind, outd, in_deg_emb, out_deg_emb, graph_token)


# speedup vs baseline: 3.9661x; 1.0199x over previous
import functools

import jax
import jax.numpy as jnp
from jax import lax
from jax.experimental import pallas as pl
from jax.experimental.pallas import tpu as pltpu
from jax.experimental.pallas import tpu_sc as plsc

G = 128
N = 512
D = 128
NC = 2
NS = 16
NW = NC * NS
GPW = G // NW
B = 128
NB = N // B
NBLK = GPW * NB
LANES = 16
SL = D // LANES


def _body(x_hbm, ind_hbm, outd_hbm, inemb_hbm, outemb_hbm, tok_hbm, out_hbm,
          idx_in, idx_out, xv, inr, outr, ov, tokv, sem_in, sem_out,
          sem_tail):
    wid = lax.axis_index("s") * NC + lax.axis_index("c")
    g0 = wid * GPW

    node_base = g0 * N
    pltpu.sync_copy(ind_hbm.at[pl.ds(node_base, GPW * N)], idx_in)
    pltpu.sync_copy(outd_hbm.at[pl.ds(node_base, GPW * N)], idx_out)
    pltpu.sync_copy(tok_hbm, tokv)

    def fetch(t, p):
        g, blk = divmod(t, NB)
        off = g * N + blk * B
        return (
            pltpu.async_copy(inemb_hbm.at[idx_in.at[pl.ds(off, B)]],
                             inr[p], sem_in[p]),
            pltpu.async_copy(outemb_hbm.at[idx_out.at[pl.ds(off, B)]],
                             outr[p], sem_in[p]),
            pltpu.async_copy(x_hbm.at[g0 + g, pl.ds(blk * B, B), :], xv[p],
                             sem_in[p]),
        )

    inflight = [None, None]
    store = None
    tail = None
    inflight[0] = fetch(0, 0)
    for t in range(NBLK):
        p = t % 2
        g, blk = divmod(t, NB)
        if t + 1 < NBLK:
            inflight[1 - p] = fetch(t + 1, 1 - p)
        for cp in inflight[p]:
            cp.wait()
        if store is not None:
            store.wait()
        if tail is not None:
            tail.wait()
            tail = None

        if blk == 0:
            for j in range(SL):
                s = pl.ds(j * LANES, LANES)
                ov[0, s] = tokv[0, s]
        else:
            for j in range(SL):
                s = pl.ds(j * LANES, LANES)
                ov[0, s] = ov[B, s]

        xvp, inrp, outrp = xv[p], inr[p], outr[p]

        @plsc.parallel_loop(0, B, unroll=4)
        def _(i):
            for j in range(SL):
                s = pl.ds(j * LANES, LANES)
                ov[i + 1, s] = xvp[i, s] + inrp[i, s] + outrp[i, s]

        store = pltpu.async_copy(
            ov.at[pl.ds(0, B)],
            out_hbm.at[g0 + g, pl.ds(blk * B, B), :], sem_out)
        if blk == NB - 1:
            tail = pltpu.async_copy(
                ov.at[pl.ds(B, 1)],
                out_hbm.at[g0 + g, pl.ds(N, 1), :], sem_tail)
    store.wait()
    tail.wait()


@jax.jit
def _run(x, ind, outd, inemb, outemb, tok):
    mesh = plsc.VectorSubcoreMesh(core_axis_name="c", subcore_axis_name="s")
    fn = functools.partial(
        pl.kernel,
        out_type=jax.ShapeDtypeStruct((G, N + 1, D), jnp.float32),
        mesh=mesh,
        scratch_types=[
            pltpu.VMEM((GPW * N,), jnp.int32),
            pltpu.VMEM((GPW * N,), jnp.int32),
            [pltpu.VMEM((B, D), jnp.float32)] * 2,
            [pltpu.VMEM((B, D), jnp.float32)] * 2,
            [pltpu.VMEM((B, D), jnp.float32)] * 2,
            pltpu.VMEM((B + 8, D), jnp.float32),
            pltpu.VMEM((1, D), jnp.float32),
            [pltpu.SemaphoreType.DMA] * 2,
            pltpu.SemaphoreType.DMA,
            pltpu.SemaphoreType.DMA,
        ],
    )(_body)
    return fn(x, ind, outd, inemb, outemb, tok)


def kernel(x, in_degree, out_degree, in_deg_emb, out_deg_emb, graph_token):
    ind = in_degree.reshape(-1).astype(jnp.int32)
    outd = out_degree.reshape(-1).astype(jnp.int32)
    return _run(x, ind, outd, in_deg_emb, out_deg_emb, graph_token)


# speedup vs baseline: 4.3009x; 1.0844x over previous
import functools

import jax
import jax.numpy as jnp
from jax import lax
from jax.experimental import pallas as pl
from jax.experimental.pallas import tpu as pltpu
from jax.experimental.pallas import tpu_sc as plsc

G = 128
N = 512
D = 128
NC = 2
NS = 16
NW = NC * NS
GPW = G // NW
B = 128
NB = N // B
NBLK = GPW * NB
LANES = 16
SL = D // LANES
NUM_DEG = 512


def _body(x_hbm, ind_hbm, outd_hbm, inemb_hbm, outemb_hbm, tok_hbm, out_hbm,
          idx_in, idx_out, xv, inr, outr, ov, tokv, sem_in, sem_out,
          sem_tail):
    wid = lax.axis_index("s") * NC + lax.axis_index("c")
    g0 = wid * GPW

    node_base = g0 * N
    pltpu.sync_copy(ind_hbm.at[pl.ds(node_base, GPW * N)], idx_in)
    pltpu.sync_copy(outd_hbm.at[pl.ds(node_base, GPW * N)], idx_out)
    pltpu.sync_copy(tok_hbm, tokv)

    def fetch(t, p):
        g, blk = divmod(t, NB)
        off = g * N + blk * B
        return (
            pltpu.async_copy(inemb_hbm.at[idx_in.at[pl.ds(off, B)]],
                             inr[p], sem_in[p]),
            pltpu.async_copy(outemb_hbm.at[idx_out.at[pl.ds(off, B)]],
                             outr[p], sem_in[p]),
            pltpu.async_copy(x_hbm.at[g0 + g, pl.ds(blk * B, B), :], xv[p],
                             sem_in[p]),
        )

    inflight = [None, None]
    store = None
    tail = None
    inflight[0] = fetch(0, 0)
    for t in range(NBLK):
        p = t % 2
        g, blk = divmod(t, NB)
        if t + 1 < NBLK:
            inflight[1 - p] = fetch(t + 1, 1 - p)
        for cp in inflight[p]:
            cp.wait()
        if store is not None:
            store.wait()
        if tail is not None:
            tail.wait()
            tail = None

        if blk == 0:
            for j in range(SL):
                s = pl.ds(j * LANES, LANES)
                ov[0, s] = tokv[0, s]
        else:
            for j in range(SL):
                s = pl.ds(j * LANES, LANES)
                ov[0, s] = ov[B, s]

        xvp, inrp, outrp = xv[p], inr[p], outr[p]

        @plsc.parallel_loop(0, B, unroll=4)
        def _(i):
            for j in range(SL):
                s = pl.ds(j * LANES, LANES)
                ov[i + 1, s] = xvp[i, s] + inrp[i, s] + outrp[i, s]

        store = pltpu.async_copy(
            ov.at[pl.ds(0, B)],
            out_hbm.at[g0 + g, pl.ds(blk * B, B), :], sem_out)
        if blk == NB - 1:
            tail = pltpu.async_copy(
                ov.at[pl.ds(B, 1)],
                out_hbm.at[g0 + g, pl.ds(N, 1), :], sem_tail)
    store.wait()
    tail.wait()


@jax.jit
def _run(x, ind, outd, inemb, outemb, tok):
    mesh = plsc.VectorSubcoreMesh(core_axis_name="c", subcore_axis_name="s")
    fn = functools.partial(
        pl.kernel,
        out_type=jax.ShapeDtypeStruct((G, N + 1, D), jnp.float32),
        mesh=mesh,
        scratch_types=[
            pltpu.VMEM((GPW * N,), jnp.int32),
            pltpu.VMEM((GPW * N,), jnp.int32),
            [pltpu.VMEM((B, D), jnp.float32)] * 2,
            [pltpu.VMEM((B, D), jnp.float32)] * 2,
            [pltpu.VMEM((B, D), jnp.float32)] * 2,
            pltpu.VMEM((B + 8, D), jnp.float32),
            pltpu.VMEM((1, D), jnp.float32),
            [pltpu.SemaphoreType.DMA] * 2,
            pltpu.SemaphoreType.DMA,
            pltpu.SemaphoreType.DMA,
        ],
    )(_body)
    return fn(x, ind, outd, inemb, outemb, tok)


def kernel(x, in_degree, out_degree, in_deg_emb, out_deg_emb, graph_token):
    shift = (jnp.arange(NW, dtype=jnp.int32) * NUM_DEG)[:, None]
    ind = (in_degree.astype(jnp.int32).reshape(NW, -1) + shift).reshape(-1)
    outd = (out_degree.astype(jnp.int32).reshape(NW, -1) + shift).reshape(-1)
    inemb = jnp.tile(in_deg_emb, (NW, 1))
    outemb = jnp.tile(out_deg_emb, (NW, 1))
    return _run(x, ind, outd, inemb, outemb, graph_token)


# speedup vs baseline: 4.4410x; 1.0326x over previous
import functools

import jax
import jax.numpy as jnp
from jax import lax
from jax.experimental import pallas as pl
from jax.experimental.pallas import tpu as pltpu
from jax.experimental.pallas import tpu_sc as plsc

G = 128
N = 512
D = 128
NC = 2
NS = 16
NW = NC * NS
GPW = G // NW
B = 128
NB = N // B
NBLK = GPW * NB
LANES = 16
SL = D // LANES
NUM_DEG = 512
NREP = 8


def _body(x_hbm, ind_hbm, outd_hbm, inemb_hbm, outemb_hbm, tok_hbm, out_hbm,
          idx_in, idx_out, xv, inr, outr, ov, tokv, sem_in, sem_out,
          sem_tail):
    wid = lax.axis_index("s") * NC + lax.axis_index("c")
    g0 = wid * GPW

    node_base = g0 * N
    pltpu.sync_copy(ind_hbm.at[pl.ds(node_base, GPW * N)], idx_in)
    pltpu.sync_copy(outd_hbm.at[pl.ds(node_base, GPW * N)], idx_out)
    pltpu.sync_copy(tok_hbm, tokv)

    def fetch(t, p):
        g, blk = divmod(t, NB)
        off = g * N + blk * B
        return (
            pltpu.async_copy(inemb_hbm.at[idx_in.at[pl.ds(off, B)]],
                             inr[p], sem_in[p]),
            pltpu.async_copy(outemb_hbm.at[idx_out.at[pl.ds(off, B)]],
                             outr[p], sem_in[p]),
            pltpu.async_copy(x_hbm.at[g0 + g, pl.ds(blk * B, B), :], xv[p],
                             sem_in[p]),
        )

    inflight = [None, None]
    store = None
    tail = None
    inflight[0] = fetch(0, 0)
    for t in range(NBLK):
        p = t % 2
        g, blk = divmod(t, NB)
        if t + 1 < NBLK:
            inflight[1 - p] = fetch(t + 1, 1 - p)
        for cp in inflight[p]:
            cp.wait()
        if store is not None:
            store.wait()
        if tail is not None:
            tail.wait()
            tail = None

        if blk == 0:
            for j in range(SL):
                s = pl.ds(j * LANES, LANES)
                ov[0, s] = tokv[0, s]
        else:
            for j in range(SL):
                s = pl.ds(j * LANES, LANES)
                ov[0, s] = ov[B, s]

        xvp, inrp, outrp = xv[p], inr[p], outr[p]

        @plsc.parallel_loop(0, B, unroll=4)
        def _(i):
            for j in range(SL):
                s = pl.ds(j * LANES, LANES)
                ov[i + 1, s] = xvp[i, s] + inrp[i, s] + outrp[i, s]

        store = pltpu.async_copy(
            ov.at[pl.ds(0, B)],
            out_hbm.at[g0 + g, pl.ds(blk * B, B), :], sem_out)
        if blk == NB - 1:
            tail = pltpu.async_copy(
                ov.at[pl.ds(B, 1)],
                out_hbm.at[g0 + g, pl.ds(N, 1), :], sem_tail)
    store.wait()
    tail.wait()


@jax.jit
def _run(x, ind, outd, inemb, outemb, tok):
    mesh = plsc.VectorSubcoreMesh(core_axis_name="c", subcore_axis_name="s")
    fn = functools.partial(
        pl.kernel,
        out_type=jax.ShapeDtypeStruct((G, N + 1, D), jnp.float32),
        mesh=mesh,
        scratch_types=[
            pltpu.VMEM((GPW * N,), jnp.int32),
            pltpu.VMEM((GPW * N,), jnp.int32),
            [pltpu.VMEM((B, D), jnp.float32)] * 2,
            [pltpu.VMEM((B, D), jnp.float32)] * 2,
            [pltpu.VMEM((B, D), jnp.float32)] * 2,
            pltpu.VMEM((B + 8, D), jnp.float32),
            pltpu.VMEM((1, D), jnp.float32),
            [pltpu.SemaphoreType.DMA] * 2,
            pltpu.SemaphoreType.DMA,
            pltpu.SemaphoreType.DMA,
        ],
    )(_body)
    return fn(x, ind, outd, inemb, outemb, tok)


def kernel(x, in_degree, out_degree, in_deg_emb, out_deg_emb, graph_token):
    shift = ((jnp.arange(NW, dtype=jnp.int32) % NREP) * NUM_DEG)[:, None]
    ind = (in_degree.astype(jnp.int32).reshape(NW, -1) + shift).reshape(-1)
    outd = (out_degree.astype(jnp.int32).reshape(NW, -1) + shift).reshape(-1)
    inemb = jnp.tile(in_deg_emb, (NREP, 1))
    outemb = jnp.tile(out_deg_emb, (NREP, 1))
    return _run(x, ind, outd, inemb, outemb, graph_token)


# speedup vs baseline: 4.4853x; 1.0100x over previous
import functools

import jax
import jax.numpy as jnp
from jax import lax
from jax.experimental import pallas as pl
from jax.experimental.pallas import tpu as pltpu
from jax.experimental.pallas import tpu_sc as plsc

G = 128
N = 512
D = 128
NC = 2
NS = 16
NW = NC * NS
GPW = G // NW
B = 128
NB = N // B
NBLK = GPW * NB
LANES = 16
SL = D // LANES
NUM_DEG = 512
NREP = 8
NOV = 3


def _body(x_hbm, ind_hbm, outd_hbm, inemb_hbm, outemb_hbm, tok_hbm, out_hbm,
          idx_in, idx_out, inr, outr, ov, tokv, sem_in, sem_out):
    wid = lax.axis_index("s") * NC + lax.axis_index("c")
    g0 = wid * GPW

    node_base = g0 * N
    pltpu.sync_copy(ind_hbm.at[pl.ds(node_base, GPW * N)], idx_in)
    pltpu.sync_copy(outd_hbm.at[pl.ds(node_base, GPW * N)], idx_out)
    pltpu.sync_copy(tok_hbm, tokv)

    def fetch(t):
        p = t % 2
        g, blk = divmod(t, NB)
        off = g * N + blk * B
        return (
            pltpu.async_copy(inemb_hbm.at[idx_in.at[pl.ds(off, B)]],
                             inr[p], sem_in[p]),
            pltpu.async_copy(outemb_hbm.at[idx_out.at[pl.ds(off, B)]],
                             outr[p], sem_in[p]),
            pltpu.async_copy(x_hbm.at[g0 + g, pl.ds(blk * B, B), :],
                             ov[t % NOV].at[pl.ds(1, B)], sem_in[p]),
        )

    inflight = [None] * NOV
    stores = [None] * NOV
    inflight[0] = fetch(0)
    for t in range(NBLK):
        p = t % 2
        b = t % NOV
        g, blk = divmod(t, NB)
        if t + 1 < NBLK:
            bn = (t + 1) % NOV
            for st in stores[bn] or ():
                st.wait()
            stores[bn] = None
            inflight[bn] = fetch(t + 1)
        for cp in inflight[b]:
            cp.wait()
        for st in stores[b] or ():
            st.wait()
        stores[b] = None

        ovb = ov[b]
        if blk == 0:
            for j in range(SL):
                s = pl.ds(j * LANES, LANES)
                ovb[0, s] = tokv[0, s]
        else:
            ovp = ov[(t - 1) % NOV]
            for j in range(SL):
                s = pl.ds(j * LANES, LANES)
                ovb[0, s] = ovp[B, s]

        inrp, outrp = inr[p], outr[p]

        @plsc.parallel_loop(0, B, unroll=4)
        def _(i):
            for j in range(SL):
                s = pl.ds(j * LANES, LANES)
                plsc.addupdate(ovb.at[i + 1, s], inrp[i, s] + outrp[i, s])

        blk_stores = [pltpu.async_copy(
            ovb.at[pl.ds(0, B)],
            out_hbm.at[g0 + g, pl.ds(blk * B, B), :], sem_out[b])]
        if blk == NB - 1:
            blk_stores.append(pltpu.async_copy(
                ovb.at[pl.ds(B, 1)],
                out_hbm.at[g0 + g, pl.ds(N, 1), :], sem_out[b]))
        stores[b] = blk_stores
    for sts in stores:
        for st in sts or ():
            st.wait()


@jax.jit
def _run(x, ind, outd, inemb, outemb, tok):
    mesh = plsc.VectorSubcoreMesh(core_axis_name="c", subcore_axis_name="s")
    fn = functools.partial(
        pl.kernel,
        out_type=jax.ShapeDtypeStruct((G, N + 1, D), jnp.float32),
        mesh=mesh,
        scratch_types=[
            pltpu.VMEM((GPW * N,), jnp.int32),
            pltpu.VMEM((GPW * N,), jnp.int32),
            [pltpu.VMEM((B, D), jnp.float32)] * 2,
            [pltpu.VMEM((B, D), jnp.float32)] * 2,
            [pltpu.VMEM((B + 8, D), jnp.float32)] * NOV,
            pltpu.VMEM((1, D), jnp.float32),
            [pltpu.SemaphoreType.DMA] * 2,
            [pltpu.SemaphoreType.DMA] * NOV,
        ],
    )(_body)
    return fn(x, ind, outd, inemb, outemb, tok)


def kernel(x, in_degree, out_degree, in_deg_emb, out_deg_emb, graph_token):
    shift = ((jnp.arange(NW, dtype=jnp.int32) % NREP) * NUM_DEG)[:, None]
    ind = (in_degree.astype(jnp.int32).reshape(NW, -1) + shift).reshape(-1)
    outd = (out_degree.astype(jnp.int32).reshape(NW, -1) + shift).reshape(-1)
    inemb = jnp.tile(in_deg_emb, (NREP, 1))
    outemb = jnp.tile(out_deg_emb, (NREP, 1))
    return _run(x, ind, outd, inemb, outemb, graph_token)
